# Initial kernel scaffold; baseline (speedup 1.0000x reference)
import jax
import jax.numpy as jnp
from jax.experimental import pallas as pl


def kernel(x, edge_index, edge_attr, We1, be1, We2, be2, eps, Wm1, bm1, gamma1, beta1, mu1, var1, Wm2, bm2, gamma2, beta2, mu2, var2):
    raise NotImplementedError("write your pallas kernel here")


# speedup vs baseline: 2.9040x; 2.9040x over previous
import functools

import jax
import jax.numpy as jnp
from jax import lax
from jax.experimental import pallas as pl
from jax.experimental.pallas import tpu as pltpu
from jax.experimental.pallas import tpu_sc as plsc

N = 10000
E = 320000
D = 128
BN_EPS = 1e-3

NC = 2
NS = 16
NW = NC * NS
EPW = E // NW
CH = 128
NFULL = EPW // CH
TAIL = EPW - NFULL * CH
WB = 640

BE = 2000
BNODE = 2000


def _he_body(ea_ref, w_ref, b_ref, out_ref):
    out_ref[...] = (
        jnp.dot(ea_ref[...], w_ref[...], preferred_element_type=jnp.float32)
        + b_ref[...]
    )


def _compute_he(edge_attr, w, b):
    return pl.pallas_call(
        _he_body,
        grid=(E // BE,),
        in_specs=[
            pl.BlockSpec((BE, 16), lambda i: (i, 0)),
            pl.BlockSpec((16, D), lambda i: (0, 0)),
            pl.BlockSpec((1, D), lambda i: (0, 0)),
        ],
        out_specs=pl.BlockSpec((BE, D), lambda i: (i, 0)),
        out_shape=jax.ShapeDtypeStruct((E, D), jnp.float32),
    )(edge_attr, w, b)


def _sc_pool_body(src_hbm, dst_hbm, he_hbm, x_hbm, out_hbm,
                  src_v, dst_v, srct_v, dstt_v, xr_v, he_v, xt_v, het_v,
                  pooled_sh, sem):
    cid = lax.axis_index("c")
    sid = lax.axis_index("s")
    wid = sid * NC + cid

    def zrow(r, carry):
        for c in range(D // 16):
            xr_v[r, pl.ds(c * 16, 16)] = jnp.zeros((16,), jnp.float32)
        return carry
    lax.fori_loop(0, CH, zrow, 0)
    for k in range(WB // CH):
        r0 = sid * WB + k * CH

        @pl.when(r0 + CH <= N)
        def _():
            pltpu.sync_copy(xr_v, pooled_sh.at[pl.ds(r0, CH)])

    @pl.when(sid == NS - 1)
    def _():
        pltpu.sync_copy(xr_v.at[pl.ds(0, N % CH)],
                        pooled_sh.at[pl.ds(N - N % CH, N % CH)])
    plsc.subcore_barrier()

    base_w = wid * EPW

    def chunk(j, carry):
        base = base_w + j * CH
        pltpu.sync_copy(src_hbm.at[pl.ds(base, CH)], src_v)
        pltpu.sync_copy(dst_hbm.at[pl.ds(base, CH)], dst_v)
        gat = pltpu.async_copy(x_hbm.at[src_v], xr_v, sem)
        pltpu.sync_copy(he_hbm.at[pl.ds(base, CH)], he_v)
        gat.wait()

        def row(r, c2):
            for c in range(D // 16):
                s = c * 16
                xr_v[r, pl.ds(s, 16)] = jnp.maximum(
                    xr_v[r, pl.ds(s, 16)] + he_v[r, pl.ds(s, 16)], 0.0)
            return c2
        lax.fori_loop(0, CH, row, 0)
        pltpu.sync_copy(xr_v, pooled_sh.at[dst_v], add=True)
        return carry
    lax.fori_loop(0, NFULL, chunk, 0)

    base = base_w + NFULL * CH
    pltpu.sync_copy(src_hbm.at[pl.ds(base, TAIL)], srct_v)
    pltpu.sync_copy(dst_hbm.at[pl.ds(base, TAIL)], dstt_v)
    gat = pltpu.async_copy(x_hbm.at[srct_v], xt_v, sem)
    pltpu.sync_copy(he_hbm.at[pl.ds(base, TAIL)], het_v)
    gat.wait()

    def trow(r, carry):
        for c in range(D // 16):
            s = c * 16
            xt_v[r, pl.ds(s, 16)] = jnp.maximum(
                xt_v[r, pl.ds(s, 16)] + het_v[r, pl.ds(s, 16)], 0.0)
        return carry
    lax.fori_loop(0, TAIL, trow, 0)
    pltpu.sync_copy(xt_v, pooled_sh.at[dstt_v], add=True)

    plsc.subcore_barrier()
    for k in range(WB // CH):
        r0 = sid * WB + k * CH

        @pl.when(r0 + CH <= N)
        def _():
            pltpu.sync_copy(pooled_sh.at[pl.ds(r0, CH)],
                            out_hbm.at[cid, pl.ds(r0, CH)])

    @pl.when(sid == NS - 1)
    def _():
        pltpu.sync_copy(pooled_sh.at[pl.ds(N - N % CH, N % CH)],
                        out_hbm.at[cid, pl.ds(N - N % CH, N % CH)])


def _sc_pool(src, dst, he, x):
    mesh = plsc.VectorSubcoreMesh(core_axis_name="c", subcore_axis_name="s")
    f = pl.kernel(
        _sc_pool_body,
        out_type=jax.ShapeDtypeStruct((NC, N, D), jnp.float32),
        mesh=mesh,
        scratch_types=[
            pltpu.VMEM((CH,), jnp.int32),
            pltpu.VMEM((CH,), jnp.int32),
            pltpu.VMEM((TAIL,), jnp.int32),
            pltpu.VMEM((TAIL,), jnp.int32),
            pltpu.VMEM((CH, D), jnp.float32),
            pltpu.VMEM((CH, D), jnp.float32),
            pltpu.VMEM((TAIL, D), jnp.float32),
            pltpu.VMEM((TAIL, D), jnp.float32),
            pltpu.VMEM_SHARED((N, D), jnp.float32),
            pltpu.SemaphoreType.DMA,
        ],
    )
    return f(src, dst, he, x)


def _node_body(eps_ref, x_ref, p_ref, w1_ref, b1_ref, w2_ref, b2_ref, out_ref):
    z = eps_ref[0, 0] * x_ref[...] + p_ref[0] + p_ref[1]
    h = jnp.maximum(
        jnp.dot(z, w1_ref[...], preferred_element_type=jnp.float32)
        + b1_ref[...], 0.0)
    out_ref[...] = (
        jnp.dot(h, w2_ref[...], preferred_element_type=jnp.float32)
        + b2_ref[...]
    )


def _node_update(epsp, x, pooled2, w1, b1, w2, b2):
    return pl.pallas_call(
        _node_body,
        grid=(N // BNODE,),
        in_specs=[
            pl.BlockSpec(memory_space=pltpu.SMEM),
            pl.BlockSpec((BNODE, D), lambda i: (i, 0)),
            pl.BlockSpec((NC, BNODE, D), lambda i: (0, i, 0)),
            pl.BlockSpec((D, 2 * D), lambda i: (0, 0)),
            pl.BlockSpec((1, 2 * D), lambda i: (0, 0)),
            pl.BlockSpec((2 * D, D), lambda i: (0, 0)),
            pl.BlockSpec((1, D), lambda i: (0, 0)),
        ],
        out_specs=pl.BlockSpec((BNODE, D), lambda i: (i, 0)),
        out_shape=jax.ShapeDtypeStruct((N, D), jnp.float32),
    )(epsp, x, pooled2, w1, b1, w2, b2)


def kernel(x, edge_index, edge_attr, We1, be1, We2, be2, eps, Wm1, bm1,
           gamma1, beta1, mu1, var1, Wm2, bm2, gamma2, beta2, mu2, var2):
    w_e = We1 @ We2
    b_e = be1 @ We2 + be2
    scale1 = gamma1 / jnp.sqrt(var1 + BN_EPS)
    w1 = Wm1 * scale1[None, :]
    b1 = (bm1 - mu1) * scale1 + beta1
    scale2 = gamma2 / jnp.sqrt(var2 + BN_EPS)
    w2 = Wm2 * scale2[None, :]
    b2 = (bm2 - mu2) * scale2 + beta2
    epsp = (1.0 + eps).reshape(1, 1)

    src = edge_index[0]
    dst = edge_index[1]

    he = _compute_he(edge_attr, w_e, b_e[None, :])
    pooled2 = _sc_pool(src, dst, he, x)
    return _node_update(epsp, x, pooled2, w1, b1[None, :], w2, b2[None, :])


# speedup vs baseline: 3.4396x; 1.1844x over previous
import functools

import jax
import jax.numpy as jnp
from jax import lax
from jax.experimental import pallas as pl
from jax.experimental.pallas import tpu as pltpu
from jax.experimental.pallas import tpu_sc as plsc

N = 10000
E = 320000
D = 128
BN_EPS = 1e-3

NC = 2
NS = 16
NW = NC * NS
EPW = E // NW
CH = 80
NFULL = EPW // CH
WB = 640

BE = 2000
BNODE = 2000


def _he_body(ea_ref, w_ref, b_ref, out_ref):
    out_ref[...] = (
        jnp.dot(ea_ref[...], w_ref[...], preferred_element_type=jnp.float32)
        + b_ref[...]
    )


def _compute_he(edge_attr, w, b):
    return pl.pallas_call(
        _he_body,
        grid=(E // BE,),
        in_specs=[
            pl.BlockSpec((BE, 16), lambda i: (i, 0)),
            pl.BlockSpec((16, D), lambda i: (0, 0)),
            pl.BlockSpec((1, D), lambda i: (0, 0)),
        ],
        out_specs=pl.BlockSpec((BE, D), lambda i: (i, 0)),
        out_shape=jax.ShapeDtypeStruct((E, D), jnp.float32),
    )(edge_attr, w, b)


def _sc_pool_body(src_hbm, dst_hbm, he_hbm, x_hbm, out_hbm,
                  src2_v, dst2_v, xr2_v, he2_v,
                  pooled_sh, gsem0, gsem1, hsem0, hsem1):
    cid = lax.axis_index("c")
    sid = lax.axis_index("s")
    wid = sid * NC + cid
    gsems = (gsem0, gsem1)
    hsems = (hsem0, hsem1)

    def zrow(r, carry):
        for c in range(D // 16):
            xr2_v[0, r, pl.ds(c * 16, 16)] = jnp.zeros((16,), jnp.float32)
        return carry
    lax.fori_loop(0, CH, zrow, 0)
    for k in range(WB // CH):
        r0 = sid * WB + k * CH

        @pl.when(r0 + CH <= N)
        def _():
            pltpu.sync_copy(xr2_v.at[0], pooled_sh.at[pl.ds(r0, CH)])

    plsc.subcore_barrier()

    base_w = wid * EPW

    def start_chunk(c, b):
        base = base_w + c * CH
        pltpu.sync_copy(src_hbm.at[pl.ds(base, CH)], src2_v.at[b])
        pltpu.sync_copy(dst_hbm.at[pl.ds(base, CH)], dst2_v.at[b])
        pltpu.async_copy(x_hbm.at[src2_v.at[b]], xr2_v.at[b], gsems[b])
        pltpu.async_copy(he_hbm.at[pl.ds(base, CH)], he2_v.at[b], hsems[b])

    def process_chunk(c, b):
        pltpu.make_async_copy(x_hbm.at[src2_v.at[b]], xr2_v.at[b],
                              gsems[b]).wait()
        pltpu.make_async_copy(he_hbm.at[pl.ds(0, CH)], he2_v.at[b],
                              hsems[b]).wait()

        def row(r, c2):
            for k in range(D // 16):
                s = k * 16
                xr2_v[b, r, pl.ds(s, 16)] = jnp.maximum(
                    xr2_v[b, r, pl.ds(s, 16)] + he2_v[b, r, pl.ds(s, 16)],
                    0.0)
            return c2
        lax.fori_loop(0, CH, row, 0)
        pltpu.sync_copy(xr2_v.at[b], pooled_sh.at[dst2_v.at[b]], add=True)

        @pl.when(c + 2 < NFULL)
        def _():
            start_chunk(c + 2, b)

    start_chunk(0, 0)
    start_chunk(1, 1)

    def pair(j2, carry):
        process_chunk(2 * j2, 0)
        process_chunk(2 * j2 + 1, 1)
        return carry
    lax.fori_loop(0, NFULL // 2, pair, 0)
    process_chunk(NFULL - 1, 0)

    plsc.subcore_barrier()
    for k in range(WB // CH):
        r0 = sid * WB + k * CH

        @pl.when(r0 + CH <= N)
        def _():
            pltpu.sync_copy(pooled_sh.at[pl.ds(r0, CH)],
                            out_hbm.at[cid, pl.ds(r0, CH)])


def _sc_pool(src, dst, he, x):
    mesh = plsc.VectorSubcoreMesh(core_axis_name="c", subcore_axis_name="s")
    f = pl.kernel(
        _sc_pool_body,
        out_type=jax.ShapeDtypeStruct((NC, N, D), jnp.float32),
        mesh=mesh,
        scratch_types=[
            pltpu.VMEM((2, CH), jnp.int32),
            pltpu.VMEM((2, CH), jnp.int32),
            pltpu.VMEM((2, CH, D), jnp.float32),
            pltpu.VMEM((2, CH, D), jnp.float32),
            pltpu.VMEM_SHARED((N, D), jnp.float32),
            pltpu.SemaphoreType.DMA,
            pltpu.SemaphoreType.DMA,
            pltpu.SemaphoreType.DMA,
            pltpu.SemaphoreType.DMA,
        ],
    )
    return f(src, dst, he, x)


def _node_body(eps_ref, x_ref, p_ref, w1_ref, b1_ref, w2_ref, b2_ref, out_ref):
    z = eps_ref[0, 0] * x_ref[...] + p_ref[0] + p_ref[1]
    h = jnp.maximum(
        jnp.dot(z, w1_ref[...], preferred_element_type=jnp.float32)
        + b1_ref[...], 0.0)
    out_ref[...] = (
        jnp.dot(h, w2_ref[...], preferred_element_type=jnp.float32)
        + b2_ref[...]
    )


def _node_update(epsp, x, pooled2, w1, b1, w2, b2):
    return pl.pallas_call(
        _node_body,
        grid=(N // BNODE,),
        in_specs=[
            pl.BlockSpec(memory_space=pltpu.SMEM),
            pl.BlockSpec((BNODE, D), lambda i: (i, 0)),
            pl.BlockSpec((NC, BNODE, D), lambda i: (0, i, 0)),
            pl.BlockSpec((D, 2 * D), lambda i: (0, 0)),
            pl.BlockSpec((1, 2 * D), lambda i: (0, 0)),
            pl.BlockSpec((2 * D, D), lambda i: (0, 0)),
            pl.BlockSpec((1, D), lambda i: (0, 0)),
        ],
        out_specs=pl.BlockSpec((BNODE, D), lambda i: (i, 0)),
        out_shape=jax.ShapeDtypeStruct((N, D), jnp.float32),
    )(epsp, x, pooled2, w1, b1, w2, b2)


def kernel(x, edge_index, edge_attr, We1, be1, We2, be2, eps, Wm1, bm1,
           gamma1, beta1, mu1, var1, Wm2, bm2, gamma2, beta2, mu2, var2):
    w_e = We1 @ We2
    b_e = be1 @ We2 + be2
    scale1 = gamma1 / jnp.sqrt(var1 + BN_EPS)
    w1 = Wm1 * scale1[None, :]
    b1 = (bm1 - mu1) * scale1 + beta1
    scale2 = gamma2 / jnp.sqrt(var2 + BN_EPS)
    w2 = Wm2 * scale2[None, :]
    b2 = (bm2 - mu2) * scale2 + beta2
    epsp = (1.0 + eps).reshape(1, 1)

    src = edge_index[0]
    dst = edge_index[1]

    he = _compute_he(edge_attr, w_e, b_e[None, :])
    pooled2 = _sc_pool(src, dst, he, x)
    return _node_update(epsp, x, pooled2, w1, b1[None, :], w2, b2[None, :])
